# Initial kernel scaffold; baseline (speedup 1.0000x reference)
import jax
import jax.numpy as jnp
from jax.experimental import pallas as pl


def kernel(z, table):
    raise NotImplementedError("write your pallas kernel here")


# speedup vs baseline: 1.5609x; 1.5609x over previous
import functools

import jax
import jax.numpy as jnp
from jax import lax
from jax.experimental import pallas as pl
from jax.experimental.pallas import tpu as pltpu
from jax.experimental.pallas import tpu_sc as plsc

_NODE_NF = 1000000
_HIDDEN = 32
_BATCH = 16384
_FIELDS = 26

_B = _BATCH * _FIELDS
_NC = 2
_NS = 16
_NW = _NC * _NS
_BPW = _B // _NW
_CHUNK = 1664
_NCHUNK = _BPW // _CHUNK

_mesh = plsc.VectorSubcoreMesh(core_axis_name="c", subcore_axis_name="s")


@functools.partial(
    pl.kernel,
    mesh=_mesh,
    out_type=jax.ShapeDtypeStruct((_B, _HIDDEN), jnp.float32),
    scratch_types=[
        pltpu.VMEM((_CHUNK,), jnp.int32),
        pltpu.VMEM((_CHUNK, _HIDDEN), jnp.float32),
        pltpu.SemaphoreType.DMA,
    ],
    compiler_params=pltpu.CompilerParams(use_tc_tiling_on_sc=False),
)
def _gather_kernel(idx_hbm, table_hbm, out_hbm, idx_v, rows_v, sem):
    wid = lax.axis_index("s") * _NC + lax.axis_index("c")
    base = wid * _BPW

    def body(i, carry):
        off = base + i * _CHUNK
        pltpu.sync_copy(idx_hbm.at[pl.ds(off, _CHUNK)], idx_v)
        pltpu.async_copy(table_hbm.at[idx_v], rows_v, sem).wait()
        pltpu.sync_copy(rows_v, out_hbm.at[pl.ds(off, _CHUNK)])
        return carry

    lax.fori_loop(0, _NCHUNK, body, 0)


def kernel(z, table):
    idx = z.reshape(-1)
    out = _gather_kernel(idx, table)
    return (out.reshape(_BATCH, _FIELDS, _HIDDEN), 0)


# speedup vs baseline: 1.5807x; 1.0127x over previous
import functools

import jax
import jax.numpy as jnp
from jax import lax
from jax.experimental import pallas as pl
from jax.experimental.pallas import tpu as pltpu
from jax.experimental.pallas import tpu_sc as plsc

_NODE_NF = 1000000
_HIDDEN = 32
_BATCH = 16384
_FIELDS = 26

_B = _BATCH * _FIELDS
_NC = 2
_NS = 16
_NW = _NC * _NS
_BPW = _B // _NW
_CHUNK = 1664
_NCHUNK = _BPW // _CHUNK

_mesh = plsc.VectorSubcoreMesh(core_axis_name="c", subcore_axis_name="s")


@functools.partial(
    pl.kernel,
    mesh=_mesh,
    out_type=jax.ShapeDtypeStruct((_B, _HIDDEN), jnp.float32),
    scratch_types=[
        pltpu.VMEM((_NCHUNK, _CHUNK), jnp.int32),
        pltpu.VMEM((2, _CHUNK, _HIDDEN), jnp.float32),
        pltpu.SemaphoreType.DMA,
        pltpu.SemaphoreType.DMA,
        pltpu.SemaphoreType.DMA,
        pltpu.SemaphoreType.DMA,
        pltpu.SemaphoreType.DMA,
    ],
    compiler_params=pltpu.CompilerParams(use_tc_tiling_on_sc=False),
)
def _gather_kernel(idx_hbm, table_hbm, out_hbm, idx_v, rows_v, sem_i,
                   sem_g0, sem_g1, sem_w0, sem_w1):
    wid = lax.axis_index("s") * _NC + lax.axis_index("c")
    base = wid * _BPW
    sem_g = (sem_g0, sem_g1)
    sem_w = (sem_w0, sem_w1)

    idx_loads = [
        pltpu.async_copy(
            idx_hbm.at[pl.ds(base + g * _CHUNK, _CHUNK)], idx_v.at[g], sem_i)
        for g in range(_NCHUNK)
    ]
    gathers = [None] * _NCHUNK
    writes = [None] * _NCHUNK

    def fire_gather(g):
        idx_loads[g].wait()
        if g >= 2:
            writes[g - 2].wait()
        gathers[g] = pltpu.async_copy(
            table_hbm.at[idx_v.at[g]], rows_v.at[g % 2], sem_g[g % 2])

    fire_gather(0)
    for g in range(_NCHUNK):
        if g + 1 < _NCHUNK:
            fire_gather(g + 1)
        gathers[g].wait()
        writes[g] = pltpu.async_copy(
            rows_v.at[g % 2], out_hbm.at[pl.ds(base + g * _CHUNK, _CHUNK)],
            sem_w[g % 2])
    writes[_NCHUNK - 2].wait()
    writes[_NCHUNK - 1].wait()


def kernel(z, table):
    idx = z.reshape(-1)
    out = _gather_kernel(idx, table)
    return (out.reshape(_BATCH, _FIELDS, _HIDDEN), 0)
